# Initial kernel scaffold; baseline (speedup 1.0000x reference)
import jax
import jax.numpy as jnp
from jax.experimental import pallas as pl


def kernel(boxes, scores):
    raise NotImplementedError("write your pallas kernel here")


# speedup vs baseline: 23.9705x; 23.9705x over previous
import functools

import jax
import jax.numpy as jnp
from jax.experimental import pallas as pl
from jax.experimental.pallas import tpu as pltpu

_PRE = 6000
_THR = 0.3
_K = 300
_NEG = float("-inf")
_LANES = 128


def _nms_body(scores_ref, coords_ref, out_ref, ws_ref, vols_ref):
    R = scores_ref.shape[0]
    scores = scores_ref[:]
    sbits = jax.lax.bitcast_convert_type(scores, jnp.int32)
    row = jax.lax.broadcasted_iota(jnp.int32, (R, _LANES), 0)
    col = jax.lax.broadcasted_iota(jnp.int32, (R, _LANES), 1)
    flat = row * _LANES + col

    def bs_val(_, lh):
        lo, hi = lh
        mid = lo + (hi - lo) // 2
        below = jnp.sum((sbits > mid).astype(jnp.int32)) < _PRE
        return (jnp.where(below, lo, mid + 1), jnp.where(below, mid, hi))

    tbits, _ = jax.lax.fori_loop(
        0, 31, bs_val, (jnp.int32(0), jnp.int32(0x3F800000)))
    gt = sbits > tbits
    eq = sbits == tbits
    need = _PRE - jnp.sum(gt.astype(jnp.int32))

    def bs_idx(_, lh):
        lo, hi = lh
        mid = lo + (hi - lo) // 2
        ok = jnp.sum((eq & (flat < mid)).astype(jnp.int32)) >= need
        return (jnp.where(ok, lo, mid + 1), jnp.where(ok, mid, hi))

    cut, _ = jax.lax.fori_loop(
        0, 16, bs_idx, (jnp.int32(0), jnp.int32(R * _LANES)))
    elig = gt | (eq & (flat < cut))

    ws_ref[:] = jnp.where(elig, scores, _NEG)
    c0 = coords_ref[0]
    c1 = coords_ref[1]
    c2 = coords_ref[2]
    c3 = coords_ref[3]
    c4 = coords_ref[4]
    c5 = coords_ref[5]
    vols_ref[:] = (c3 - c0) * (c4 - c1) * (c5 - c2)

    big = jnp.int32(2 ** 30)
    m0 = jnp.max(scores)
    idx0 = jnp.min(jnp.where(scores == m0, flat, big))
    lane = jax.lax.broadcasted_iota(jnp.int32, (1, _LANES), 1)

    def body(i, _):
        ws = ws_ref[:]
        m = jnp.max(ws)
        empty = m == _NEG
        sel = jnp.min(jnp.where(ws == m, flat, big))
        sel = jnp.where(empty, idx0, sel)
        onehot = flat == sel

        def pick(arr):
            return jnp.sum(jnp.where(onehot, arr, 0.0))

        y1 = pick(c0)
        x1 = pick(c1)
        z1 = pick(c2)
        y2 = pick(c3)
        x2 = pick(c4)
        z2 = pick(c5)
        sc = pick(scores)

        inter = (jnp.maximum(jnp.minimum(y2, c3) - jnp.maximum(y1, c0), 0.0)
                 * jnp.maximum(jnp.minimum(x2, c4) - jnp.maximum(x1, c1), 0.0)
                 * jnp.maximum(jnp.minimum(z2, c5) - jnp.maximum(z1, c2), 0.0))
        vol1 = (y2 - y1) * (x2 - x1) * (z2 - z1)
        union = vol1 + vols_ref[:] - inter
        iou = inter / (union + 1e-8)
        ws_ref[:] = jnp.where(onehot | (iou > _THR), _NEG, ws)

        v = jnp.full((1, _LANES), 0.0, jnp.float32)
        for j, val in enumerate((y1, x1, z1, y2, x2, z2, sc)):
            v = jnp.where(lane == j, val, v)
        out_ref[pl.ds(i, 1), :] = v
        return 0

    jax.lax.fori_loop(0, _K, body, 0)


@jax.jit
def kernel(boxes, scores):
    n = scores.shape[0]
    r = (n + _LANES - 1) // _LANES
    r = (r + 7) // 8 * 8
    pad = r * _LANES - n
    scores_p = jnp.concatenate(
        [scores, jnp.full((pad,), _NEG, jnp.float32)]).reshape(r, _LANES)
    boxes_p = jnp.concatenate([boxes, jnp.zeros((pad, 6), jnp.float32)], axis=0)
    coords = boxes_p.T.reshape(6, r, _LANES)

    out = pl.pallas_call(
        _nms_body,
        out_shape=jax.ShapeDtypeStruct((304, _LANES), jnp.float32),
        scratch_shapes=[
            pltpu.VMEM((r, _LANES), jnp.float32),
            pltpu.VMEM((r, _LANES), jnp.float32),
        ],
    )(scores_p, coords)
    return out[:_K, :7]


# speedup vs baseline: 26.0465x; 1.0866x over previous
import functools

import jax
import jax.numpy as jnp
from jax.experimental import pallas as pl
from jax.experimental.pallas import tpu as pltpu

_PRE = 6000
_THR = 0.3
_K = 300
_NEG = float("-inf")
_LANES = 128
_SUB = 8


def _nms_body(scores_ref, coords_ref, out_ref, ws_ref, vols_ref):
    R = scores_ref.shape[0]
    NCH = R // _SUB
    scores = scores_ref[:]
    sbits = jax.lax.bitcast_convert_type(scores, jnp.int32)
    row = jax.lax.broadcasted_iota(jnp.int32, (R, _LANES), 0)
    col = jax.lax.broadcasted_iota(jnp.int32, (R, _LANES), 1)
    flat = row * _LANES + col
    big = jnp.int32(2 ** 30)

    def bs_val(_, lh):
        lo, hi = lh
        mid = lo + (hi - lo) // 2
        below = jnp.sum((sbits > mid).astype(jnp.int32)) < _PRE
        return (jnp.where(below, lo, mid + 1), jnp.where(below, mid, hi))

    tbits, _ = jax.lax.fori_loop(
        0, 31, bs_val, (jnp.int32(0), jnp.int32(0x3F800000)))
    gt = sbits > tbits
    eq = sbits == tbits
    need = _PRE - jnp.sum(gt.astype(jnp.int32))

    def bs_idx(_, lh):
        lo, hi = lh
        mid = lo + (hi - lo) // 2
        ok = jnp.sum((eq & (flat < mid)).astype(jnp.int32)) >= need
        return (jnp.where(ok, lo, mid + 1), jnp.where(ok, mid, hi))

    cut, _ = jax.lax.fori_loop(
        0, 16, bs_idx, (jnp.int32(0), jnp.int32(R * _LANES)))
    elig = gt | (eq & (flat < cut))

    ws_ref[:] = jnp.where(elig, scores, _NEG)
    vols_ref[:] = ((coords_ref[3] - coords_ref[0])
                   * (coords_ref[4] - coords_ref[1])
                   * (coords_ref[5] - coords_ref[2]))

    ii = (jax.lax.broadcasted_iota(jnp.int32, (_SUB, _LANES), 0) * _LANES
          + jax.lax.broadcasted_iota(jnp.int32, (_SUB, _LANES), 1))
    lane = jax.lax.broadcasted_iota(jnp.int32, (1, _LANES), 1)

    def comb_ordered(acc, v, i):
        if acc is None:
            return (v, i)
        av, ai = acc
        return (jnp.maximum(av, v), jnp.where(av >= v, ai, i))

    def comb_lex(a, b):
        av, ai = a
        bv, bi = b
        c = (av > bv) | ((av == bv) & (ai < bi))
        return (jnp.where(c, av, bv), jnp.where(c, ai, bi))

    def merge(accs):
        return comb_lex(comb_lex(accs[0], accs[1]), comb_lex(accs[2], accs[3]))

    accs = [None] * 4
    for c in range(NCH):
        w = ws_ref[pl.ds(c * _SUB, _SUB), :]
        accs[c & 3] = comb_ordered(accs[c & 3], w, ii + c * _SUB * _LANES)
    v0, i0 = merge(accs)
    m0 = jnp.max(v0)
    idx0 = jnp.min(jnp.where(v0 == m0, i0, big))

    def body(i, carry):
        v8, i8 = carry
        m = jnp.max(v8)
        empty = m == _NEG
        sel = jnp.min(jnp.where(v8 == m, i8, big))
        sel = jnp.where(empty, idx0, sel)
        sc = jnp.where(empty, m0, m)
        rrow = jax.lax.shift_right_logical(sel, 7)
        lsel = jax.lax.bitwise_and(sel, 127)

        def pick(k):
            rowv = coords_ref[k, pl.ds(rrow, 1), :]
            return jnp.sum(jnp.where(lane == lsel, rowv, 0.0))

        y1 = pick(0)
        x1 = pick(1)
        z1 = pick(2)
        y2 = pick(3)
        x2 = pick(4)
        z2 = pick(5)
        vol1 = (y2 - y1) * (x2 - x1) * (z2 - z1)

        accs = [None] * 4
        for c in range(NCH):
            sl = pl.ds(c * _SUB, _SUB)
            w = ws_ref[sl, :]
            b0 = coords_ref[0, sl, :]
            b1 = coords_ref[1, sl, :]
            b2 = coords_ref[2, sl, :]
            b3 = coords_ref[3, sl, :]
            b4 = coords_ref[4, sl, :]
            b5 = coords_ref[5, sl, :]
            vv = vols_ref[sl, :]
            inter = (jnp.maximum(jnp.minimum(y2, b3) - jnp.maximum(y1, b0), 0.0)
                     * jnp.maximum(jnp.minimum(x2, b4) - jnp.maximum(x1, b1), 0.0)
                     * jnp.maximum(jnp.minimum(z2, b5) - jnp.maximum(z1, b2), 0.0))
            iou = inter / ((vol1 + vv - inter) + 1e-8)
            neww = jnp.where(iou > _THR, _NEG, w)
            ws_ref[sl, :] = neww
            accs[c & 3] = comb_ordered(accs[c & 3], neww, ii + c * _SUB * _LANES)

        v = jnp.full((1, _LANES), 0.0, jnp.float32)
        for j, val in enumerate((y1, x1, z1, y2, x2, z2, sc)):
            v = jnp.where(lane == j, val, v)
        out_ref[pl.ds(i, 1), :] = v
        return merge(accs)

    jax.lax.fori_loop(0, _K, body, (v0, i0))


@jax.jit
def kernel(boxes, scores):
    n = scores.shape[0]
    r = (n + _LANES - 1) // _LANES
    r = (r + 7) // 8 * 8
    pad = r * _LANES - n
    scores_p = jnp.concatenate(
        [scores, jnp.full((pad,), _NEG, jnp.float32)]).reshape(r, _LANES)
    boxes_p = jnp.concatenate([boxes, jnp.zeros((pad, 6), jnp.float32)], axis=0)
    coords = boxes_p.T.reshape(6, r, _LANES)

    out = pl.pallas_call(
        _nms_body,
        out_shape=jax.ShapeDtypeStruct((304, _LANES), jnp.float32),
        scratch_shapes=[
            pltpu.VMEM((r, _LANES), jnp.float32),
            pltpu.VMEM((r, _LANES), jnp.float32),
        ],
    )(scores_p, coords)
    return out[:_K, :7]


# speedup vs baseline: 27.8868x; 1.0707x over previous
import functools

import jax
import jax.numpy as jnp
from jax.experimental import pallas as pl
from jax.experimental.pallas import tpu as pltpu

_PRE = 6000
_THR = 0.3
_K = 300
_NEG = float("-inf")
_LANES = 128
_SUB = 8


def _nms_body(scores_ref, coords_ref, out_ref, ws_ref, vols_ref):
    R = scores_ref.shape[0]
    NCH = R // _SUB
    scores = scores_ref[:]
    sbits = jax.lax.bitcast_convert_type(scores, jnp.int32)
    row = jax.lax.broadcasted_iota(jnp.int32, (R, _LANES), 0)
    col = jax.lax.broadcasted_iota(jnp.int32, (R, _LANES), 1)
    flat = row * _LANES + col
    big = jnp.int32(2 ** 30)

    def bs_val(_, lh):
        lo, hi = lh
        mid = lo + (hi - lo) // 2
        below = jnp.sum((sbits > mid).astype(jnp.int32)) < _PRE
        return (jnp.where(below, lo, mid + 1), jnp.where(below, mid, hi))

    tbits, _ = jax.lax.fori_loop(
        0, 31, bs_val, (jnp.int32(0), jnp.int32(0x3F800000)))
    gt = sbits > tbits
    eq = sbits == tbits
    need = _PRE - jnp.sum(gt.astype(jnp.int32))

    def bs_idx(_, lh):
        lo, hi = lh
        mid = lo + (hi - lo) // 2
        ok = jnp.sum((eq & (flat < mid)).astype(jnp.int32)) >= need
        return (jnp.where(ok, lo, mid + 1), jnp.where(ok, mid, hi))

    cut, _ = jax.lax.fori_loop(
        0, 16, bs_idx, (jnp.int32(0), jnp.int32(R * _LANES)))
    elig = gt | (eq & (flat < cut))

    ws_ref[:] = jnp.where(elig, scores, _NEG)
    vols_ref[:] = ((coords_ref[3] - coords_ref[0])
                   * (coords_ref[4] - coords_ref[1])
                   * (coords_ref[5] - coords_ref[2]))

    ii = (jax.lax.broadcasted_iota(jnp.int32, (_SUB, _LANES), 0) * _LANES
          + jax.lax.broadcasted_iota(jnp.int32, (_SUB, _LANES), 1))
    lane = jax.lax.broadcasted_iota(jnp.int32, (1, _LANES), 1)

    def comb_ordered(acc, v, i):
        if acc is None:
            return (v, i)
        av, ai = acc
        return (jnp.maximum(av, v), jnp.where(av >= v, ai, i))

    def comb_lex(a, b):
        av, ai = a
        bv, bi = b
        c = (av > bv) | ((av == bv) & (ai < bi))
        return (jnp.where(c, av, bv), jnp.where(c, ai, bi))

    def merge(accs):
        return comb_lex(comb_lex(accs[0], accs[1]), comb_lex(accs[2], accs[3]))

    accs = [None] * 4
    for c in range(NCH):
        w = ws_ref[pl.ds(c * _SUB, _SUB), :]
        accs[c & 3] = comb_ordered(accs[c & 3], w, ii + c * _SUB * _LANES)
    v0, i0 = merge(accs)
    m0 = jnp.max(v0)
    idx0 = jnp.min(jnp.where(v0 == m0, i0, big))

    def body(i, carry):
        v8, i8 = carry
        m = jnp.max(v8)
        empty = m == _NEG
        sel = jnp.min(jnp.where(v8 == m, i8, big))
        sel = jnp.where(empty, idx0, sel)
        sc = jnp.where(empty, m0, m)
        rrow = jax.lax.shift_right_logical(sel, 7)
        lsel = jax.lax.bitwise_and(sel, 127)
        lmask = lane == lsel

        def pick(k):
            rowv = coords_ref[k, pl.ds(rrow, 1), :]
            return jnp.max(jnp.where(lmask, rowv, _NEG),
                           axis=1, keepdims=True)

        y1 = pick(0)
        x1 = pick(1)
        z1 = pick(2)
        y2 = pick(3)
        x2 = pick(4)
        z2 = pick(5)
        vol1 = (y2 - y1) * (x2 - x1) * (z2 - z1)

        accs = [None] * 4
        for c in range(NCH):
            sl = pl.ds(c * _SUB, _SUB)
            w = ws_ref[sl, :]
            b0 = coords_ref[0, sl, :]
            b1 = coords_ref[1, sl, :]
            b2 = coords_ref[2, sl, :]
            b3 = coords_ref[3, sl, :]
            b4 = coords_ref[4, sl, :]
            b5 = coords_ref[5, sl, :]
            vv = vols_ref[sl, :]
            inter = (jnp.maximum(jnp.minimum(y2, b3) - jnp.maximum(y1, b0), 0.0)
                     * jnp.maximum(jnp.minimum(x2, b4) - jnp.maximum(x1, b1), 0.0)
                     * jnp.maximum(jnp.minimum(z2, b5) - jnp.maximum(z1, b2), 0.0))
            iou = inter / ((vol1 + vv - inter) + 1e-8)
            neww = jnp.where(iou > _THR, _NEG, w)
            ws_ref[sl, :] = neww
            accs[c & 3] = comb_ordered(accs[c & 3], neww, ii + c * _SUB * _LANES)

        v = jnp.full((1, _LANES), 0.0, jnp.float32)
        for j, val in enumerate((y1, x1, z1, y2, x2, z2, sc)):
            v = jnp.where(lane == j, val, v)
        out_ref[pl.ds(i, 1), :] = v
        return merge(accs)

    jax.lax.fori_loop(0, _K, body, (v0, i0))


@jax.jit
def kernel(boxes, scores):
    n = scores.shape[0]
    r = (n + _LANES - 1) // _LANES
    r = (r + 7) // 8 * 8
    pad = r * _LANES - n
    scores_p = jnp.concatenate(
        [scores, jnp.full((pad,), _NEG, jnp.float32)]).reshape(r, _LANES)
    boxes_p = jnp.concatenate([boxes, jnp.zeros((pad, 6), jnp.float32)], axis=0)
    coords = boxes_p.T.reshape(6, r, _LANES)

    out = pl.pallas_call(
        _nms_body,
        out_shape=jax.ShapeDtypeStruct((304, _LANES), jnp.float32),
        scratch_shapes=[
            pltpu.VMEM((r, _LANES), jnp.float32),
            pltpu.VMEM((r, _LANES), jnp.float32),
        ],
    )(scores_p, coords)
    return out[:_K, :7]
